# baseline (speedup 1.0000x reference)
import functools
import math

import jax
import jax.numpy as jnp
from jax.experimental import pallas as pl
from jax.experimental.pallas import tpu as pltpu

B = 1
S = 2048
D = 2048
H = 16
HD = D // H
U = 38
UP = 40
INV_SQRT_D = 1.0 / math.sqrt(HD)
QBLK = 512
SBLK = 256


def _proj_kernel(q_ref, k_ref, wq_ref, wk_ref, bq_ref, bk_ref, qh_ref, kh_ref):
    dn = (((1,), (1,)), ((), ()))
    qh_ref[0] = (
        jax.lax.dot_general(q_ref[...], wq_ref[...], dn,
                            preferred_element_type=jnp.float32)
        + bq_ref[0]
    )
    kh_ref[0] = (
        jax.lax.dot_general(k_ref[...], wk_ref[...], dn,
                            preferred_element_type=jnp.float32)
        + bk_ref[0]
    )


def _project_qk(q2d, k2d, Wq, bq, Wk, bk):
    bq3 = bq.reshape(H, 1, HD)
    bk3 = bk.reshape(H, 1, HD)
    return pl.pallas_call(
        _proj_kernel,
        grid=(H, S // SBLK),
        in_specs=[
            pl.BlockSpec((SBLK, D), lambda h, i: (i, 0)),
            pl.BlockSpec((SBLK, D), lambda h, i: (i, 0)),
            pl.BlockSpec((HD, D), lambda h, i: (h, 0)),
            pl.BlockSpec((HD, D), lambda h, i: (h, 0)),
            pl.BlockSpec((1, 1, HD), lambda h, i: (h, 0, 0)),
            pl.BlockSpec((1, 1, HD), lambda h, i: (h, 0, 0)),
        ],
        out_specs=[
            pl.BlockSpec((1, SBLK, HD), lambda h, i: (h, i, 0)),
            pl.BlockSpec((1, SBLK, HD), lambda h, i: (h, i, 0)),
        ],
        out_shape=[
            jax.ShapeDtypeStruct((H, S, HD), jnp.float32),
            jax.ShapeDtypeStruct((H, S, HD), jnp.float32),
        ],
    )(q2d, k2d, Wq, Wk, bq3, bk3)


def _imp_kernel(qh_ref, kh_ref, imp_ref):
    k = kh_ref[0]
    dn = (((1,), (1,)), ((), ()))
    for j in range(S // QBLK):
        qblk = qh_ref[0, j * QBLK:(j + 1) * QBLK, :]
        s = jax.lax.dot_general(qblk, k, dn,
                                preferred_element_type=jnp.float32)
        s = s * INV_SQRT_D
        imp = jnp.max(s, axis=1) - jnp.mean(s, axis=1)
        imp_ref[0, 0, j * QBLK:(j + 1) * QBLK] = imp


def _importance(Qh, Kh):
    out = pl.pallas_call(
        _imp_kernel,
        grid=(H,),
        in_specs=[
            pl.BlockSpec((1, S, HD), lambda h: (h, 0, 0)),
            pl.BlockSpec((1, S, HD), lambda h: (h, 0, 0)),
        ],
        out_specs=pl.BlockSpec((1, 1, S), lambda h: (h, 0, 0)),
        out_shape=jax.ShapeDtypeStruct((H, 1, S), jnp.float32),
    )(Qh, Kh)
    return out.reshape(H, S)


def _topk_kernel(imp_ref, idx_ref):
    imp0 = imp_ref[...]
    iota = jax.lax.broadcasted_iota(jnp.int32, (H, S), 1)
    ucol = jax.lax.broadcasted_iota(jnp.int32, (H, UP), 1)

    def body(u, carry):
        imp, acc = carry
        m = jnp.max(imp, axis=1, keepdims=True)
        cand = jnp.where(imp == m, iota, S)
        idx = jnp.min(cand, axis=1, keepdims=True)
        acc = jnp.where(ucol == u, idx, acc)
        imp = jnp.where(iota == idx, -jnp.inf, imp)
        return imp, acc

    acc0 = jnp.full((H, UP), -1, jnp.int32)
    _, acc = jax.lax.fori_loop(0, U, body, (imp0, acc0))
    idx_ref[...] = acc


def _topk(imp):
    return pl.pallas_call(
        _topk_kernel,
        out_shape=jax.ShapeDtypeStruct((H, UP), jnp.int32),
    )(imp)


def _selattn_kernel(qh_ref, kh_ref, idx_ref, v_ref, wv_ref, bv_ref,
                    corr_ref, dflt_ref, vmean_ref):
    h = pl.program_id(0)

    @pl.when(h == 0)
    def _():
        vmean_ref[...] = jnp.sum(v_ref[...], axis=0, keepdims=True) * (1.0 / S)

    dn = (((1,), (1,)), ((), ()))
    q = qh_ref[0]
    k = kh_ref[0]
    idx = idx_ref[0]
    oh = (jax.lax.broadcasted_iota(jnp.int32, (UP, S), 1) == idx)
    oh = oh.astype(jnp.float32)
    qsel = jnp.dot(oh, q, preferred_element_type=jnp.float32)
    s = jax.lax.dot_general(qsel, k, dn,
                            preferred_element_type=jnp.float32) * INV_SQRT_D
    m = jnp.max(s, axis=1, keepdims=True)
    e = jnp.exp(s - m)
    denom = jnp.sum(e, axis=1, keepdims=True) + 1e-8
    w = e / denom
    wv = jnp.dot(w, v_ref[...], preferred_element_type=jnp.float32)
    osel = jax.lax.dot_general(wv, wv_ref[...], dn,
                               preferred_element_type=jnp.float32)
    wsum = jnp.sum(w, axis=1, keepdims=True)
    osel = osel + wsum * bv_ref[0]
    dflt = (jax.lax.dot_general(vmean_ref[...], wv_ref[...], dn,
                                preferred_element_type=jnp.float32)
            + bv_ref[0])
    corr_ref[0] = osel - dflt
    dflt_ref[0] = dflt


def _selected_attention(Qh, Kh, idx, v2d, Wv, bv):
    idx3 = idx.reshape(H, UP, 1)
    bv3 = bv.reshape(H, 1, HD)
    return pl.pallas_call(
        _selattn_kernel,
        grid=(H,),
        in_specs=[
            pl.BlockSpec((1, S, HD), lambda h: (h, 0, 0)),
            pl.BlockSpec((1, S, HD), lambda h: (h, 0, 0)),
            pl.BlockSpec((1, UP, 1), lambda h: (h, 0, 0)),
            pl.BlockSpec((S, D), lambda h: (0, 0)),
            pl.BlockSpec((HD, D), lambda h: (h, 0)),
            pl.BlockSpec((1, 1, HD), lambda h: (h, 0, 0)),
        ],
        out_specs=[
            pl.BlockSpec((1, UP, HD), lambda h: (h, 0, 0)),
            pl.BlockSpec((1, 1, HD), lambda h: (h, 0, 0)),
        ],
        out_shape=[
            jax.ShapeDtypeStruct((H, UP, HD), jnp.float32),
            jax.ShapeDtypeStruct((H, 1, HD), jnp.float32),
        ],
        scratch_shapes=[pltpu.VMEM((1, D), jnp.float32)],
        compiler_params=pltpu.CompilerParams(
            dimension_semantics=("arbitrary",)),
    )(Qh, Kh, idx3, v2d, Wv, bv3)


def _assemble_kernel(idx_sref, corr_ref, dflt_ref, wo_ref, bo_ref,
                     out_ref, base_ref):
    h = pl.program_id(0)
    dn = (((1,), (1,)), ((), ()))

    @pl.when(h == 0)
    def _():
        out_ref[...] = jnp.zeros((S, D), jnp.float32)
        base_ref[...] = bo_ref[...]

    row = jax.lax.dot_general(dflt_ref[0], wo_ref[...], dn,
                              preferred_element_type=jnp.float32)
    base_ref[...] += row
    corr_out = jax.lax.dot_general(corr_ref[0], wo_ref[...], dn,
                                   preferred_element_type=jnp.float32)
    for i in range(U):
        r = idx_sref[h * UP + i]
        out_ref[pl.ds(r, 1), :] += corr_out[i:i + 1, :]

    @pl.when(h == H - 1)
    def _():
        out_ref[...] += base_ref[...]


def _assemble(idx, corr, dflt, Wo, bo):
    idx_flat = idx.reshape(H * UP)
    bo2 = bo.reshape(1, D)
    grid_spec = pltpu.PrefetchScalarGridSpec(
        num_scalar_prefetch=1,
        grid=(H,),
        in_specs=[
            pl.BlockSpec((1, UP, HD), lambda h, sref: (h, 0, 0)),
            pl.BlockSpec((1, 1, HD), lambda h, sref: (h, 0, 0)),
            pl.BlockSpec((D, HD), lambda h, sref: (0, h)),
            pl.BlockSpec((1, D), lambda h, sref: (0, 0)),
        ],
        out_specs=pl.BlockSpec((S, D), lambda h, sref: (0, 0)),
        scratch_shapes=[pltpu.VMEM((1, D), jnp.float32)],
    )
    return pl.pallas_call(
        _assemble_kernel,
        grid_spec=grid_spec,
        out_shape=jax.ShapeDtypeStruct((S, D), jnp.float32),
        compiler_params=pltpu.CompilerParams(
            dimension_semantics=("arbitrary",)),
    )(idx_flat, corr, dflt, Wo, bo2)


@jax.jit
def kernel(queries, keys, values, Wq, bq, Wk, bk, Wv, bv, Wo, bo):
    q2d = queries.reshape(S, D)
    k2d = keys.reshape(S, D)
    v2d = values.reshape(S, D)
    Qh, Kh = _project_qk(q2d, k2d, Wq, bq, Wk, bk)
    imp = _importance(Qh, Kh)
    idx = _topk(imp)
    corr, dflt = _selected_attention(Qh, Kh, idx, v2d, Wv, bv)
    out = _assemble(idx, corr, dflt, Wo, bo)
    return out.reshape(B, S, D)


# speedup vs baseline: 1.7784x; 1.7784x over previous
import functools
import math

import jax
import jax.numpy as jnp
from jax.experimental import pallas as pl
from jax.experimental.pallas import tpu as pltpu

B = 1
S = 2048
D = 2048
H = 16
HD = D // H
U = 38
UP = 40
INV_SQRT_D = 1.0 / math.sqrt(HD)
QBLK = 512
SBLK = 256


def _proj_kernel(x_ref, w_ref, b_ref, o_ref):
    dn = (((1,), (1,)), ((), ()))
    o_ref[...] = (
        jax.lax.dot_general(x_ref[...], w_ref[...], dn,
                            preferred_element_type=jnp.float32)
        + b_ref[...]
    )


def _project(x2d, W, b):
    return pl.pallas_call(
        _proj_kernel,
        grid=(S // SBLK,),
        in_specs=[
            pl.BlockSpec((SBLK, D), lambda i: (i, 0)),
            pl.BlockSpec((D, D), lambda i: (0, 0)),
            pl.BlockSpec((1, D), lambda i: (0, 0)),
        ],
        out_specs=pl.BlockSpec((SBLK, D), lambda i: (i, 0)),
        out_shape=jax.ShapeDtypeStruct((S, D), jnp.float32),
    )(x2d, W, b.reshape(1, D))


def _imp_kernel(qh_ref, kh_ref, imp_ref):
    k = kh_ref[...]
    dn = (((1,), (1,)), ((), ()))
    for j in range(S // QBLK):
        qblk = qh_ref[j * QBLK:(j + 1) * QBLK, :]
        s = jax.lax.dot_general(qblk, k, dn,
                                preferred_element_type=jnp.float32)
        s = s * INV_SQRT_D
        imp = jnp.max(s, axis=1) - jnp.mean(s, axis=1)
        imp_ref[0, 0, j * QBLK:(j + 1) * QBLK] = imp


def _importance(Q2d, K2d):
    out = pl.pallas_call(
        _imp_kernel,
        grid=(H,),
        in_specs=[
            pl.BlockSpec((S, HD), lambda h: (0, h)),
            pl.BlockSpec((S, HD), lambda h: (0, h)),
        ],
        out_specs=pl.BlockSpec((1, 1, S), lambda h: (h, 0, 0)),
        out_shape=jax.ShapeDtypeStruct((H, 1, S), jnp.float32),
    )(Q2d, K2d)
    return out.reshape(H, S)


def _topk_kernel(imp_ref, idx_ref):
    imp0 = imp_ref[...]
    iota = jax.lax.broadcasted_iota(jnp.int32, (H, S), 1)
    ucol = jax.lax.broadcasted_iota(jnp.int32, (H, UP), 1)

    def body(u, carry):
        imp, acc = carry
        m = jnp.max(imp, axis=1, keepdims=True)
        cand = jnp.where(imp == m, iota, S)
        idx = jnp.min(cand, axis=1, keepdims=True)
        acc = jnp.where(ucol == u, idx, acc)
        imp = jnp.where(iota == idx, -jnp.inf, imp)
        return imp, acc

    acc0 = jnp.full((H, UP), -1, jnp.int32)
    _, acc = jax.lax.fori_loop(0, U, body, (imp0, acc0))
    idx_ref[...] = acc


def _topk(imp):
    return pl.pallas_call(
        _topk_kernel,
        out_shape=jax.ShapeDtypeStruct((H, UP), jnp.int32),
    )(imp)


def _selattn_kernel(qh_ref, kh_ref, idx_ref, v_ref, wv_ref, bv_ref,
                    corr_ref, dflt_ref, vmean_ref):
    h = pl.program_id(0)

    @pl.when(h == 0)
    def _():
        vmean_ref[...] = jnp.sum(v_ref[...], axis=0, keepdims=True) * (1.0 / S)

    dn = (((1,), (1,)), ((), ()))
    q = qh_ref[...]
    k = kh_ref[...]
    idx = idx_ref[0]
    oh = (jax.lax.broadcasted_iota(jnp.int32, (UP, S), 1) == idx)
    oh = oh.astype(jnp.float32)
    qsel = jnp.dot(oh, q, preferred_element_type=jnp.float32)
    s = jax.lax.dot_general(qsel, k, dn,
                            preferred_element_type=jnp.float32) * INV_SQRT_D
    m = jnp.max(s, axis=1, keepdims=True)
    e = jnp.exp(s - m)
    denom = jnp.sum(e, axis=1, keepdims=True) + 1e-8
    w = e / denom
    wv = jnp.dot(w, v_ref[...], preferred_element_type=jnp.float32)
    osel = jax.lax.dot_general(wv, wv_ref[...], dn,
                               preferred_element_type=jnp.float32)
    wsum = jnp.sum(w, axis=1, keepdims=True)
    osel = osel + wsum * bv_ref[0]
    dflt = (jax.lax.dot_general(vmean_ref[...], wv_ref[...], dn,
                                preferred_element_type=jnp.float32)
            + bv_ref[0])
    corr_ref[0] = osel - dflt
    dflt_ref[0] = dflt


def _selected_attention(Q2d, K2d, idx, v2d, Wv, bv):
    idx3 = idx.reshape(H, UP, 1)
    bv3 = bv.reshape(H, 1, HD)
    return pl.pallas_call(
        _selattn_kernel,
        grid=(H,),
        in_specs=[
            pl.BlockSpec((S, HD), lambda h: (0, h)),
            pl.BlockSpec((S, HD), lambda h: (0, h)),
            pl.BlockSpec((1, UP, 1), lambda h: (h, 0, 0)),
            pl.BlockSpec((S, D), lambda h: (0, 0)),
            pl.BlockSpec((HD, D), lambda h: (h, 0)),
            pl.BlockSpec((1, 1, HD), lambda h: (h, 0, 0)),
        ],
        out_specs=[
            pl.BlockSpec((1, UP, HD), lambda h: (h, 0, 0)),
            pl.BlockSpec((1, 1, HD), lambda h: (h, 0, 0)),
        ],
        out_shape=[
            jax.ShapeDtypeStruct((H, UP, HD), jnp.float32),
            jax.ShapeDtypeStruct((H, 1, HD), jnp.float32),
        ],
        scratch_shapes=[pltpu.VMEM((1, D), jnp.float32)],
        compiler_params=pltpu.CompilerParams(
            dimension_semantics=("arbitrary",)),
    )(Q2d, K2d, idx3, v2d, Wv, bv3)


def _assemble_kernel(idx_sref, corr_ref, dflt_ref, wo_ref, bo_ref,
                     out_ref, base_ref):
    h = pl.program_id(0)
    dn = (((1,), (1,)), ((), ()))

    @pl.when(h == 0)
    def _():
        out_ref[...] = jnp.zeros((S, D), jnp.float32)
        base_ref[...] = bo_ref[...]

    row = jax.lax.dot_general(dflt_ref[0], wo_ref[...], dn,
                              preferred_element_type=jnp.float32)
    base_ref[...] += row
    corr_out = jax.lax.dot_general(corr_ref[0], wo_ref[...], dn,
                                   preferred_element_type=jnp.float32)
    for i in range(U):
        r = idx_sref[h * UP + i]
        out_ref[pl.ds(r, 1), :] += corr_out[i:i + 1, :]

    @pl.when(h == H - 1)
    def _():
        out_ref[...] += base_ref[...]


def _assemble(idx, corr, dflt, Wo, bo):
    idx_flat = idx.reshape(H * UP)
    bo2 = bo.reshape(1, D)
    grid_spec = pltpu.PrefetchScalarGridSpec(
        num_scalar_prefetch=1,
        grid=(H,),
        in_specs=[
            pl.BlockSpec((1, UP, HD), lambda h, sref: (h, 0, 0)),
            pl.BlockSpec((1, 1, HD), lambda h, sref: (h, 0, 0)),
            pl.BlockSpec((D, HD), lambda h, sref: (0, h)),
            pl.BlockSpec((1, D), lambda h, sref: (0, 0)),
        ],
        out_specs=pl.BlockSpec((S, D), lambda h, sref: (0, 0)),
        scratch_shapes=[pltpu.VMEM((1, D), jnp.float32)],
    )
    return pl.pallas_call(
        _assemble_kernel,
        grid_spec=grid_spec,
        out_shape=jax.ShapeDtypeStruct((S, D), jnp.float32),
        compiler_params=pltpu.CompilerParams(
            dimension_semantics=("arbitrary",)),
    )(idx_flat, corr, dflt, Wo, bo2)


@jax.jit
def kernel(queries, keys, values, Wq, bq, Wk, bk, Wv, bv, Wo, bo):
    q2d = queries.reshape(S, D)
    k2d = keys.reshape(S, D)
    v2d = values.reshape(S, D)
    Q2d = _project(q2d, Wq, bq)
    K2d = _project(k2d, Wk, bk)
    imp = _importance(Q2d, K2d)
    idx = _topk(imp)
    corr, dflt = _selected_attention(Q2d, K2d, idx, v2d, Wv, bv)
    out = _assemble(idx, corr, dflt, Wo, bo)
    return out.reshape(B, S, D)
